# speedup vs baseline: 4.2045x; 4.2045x over previous
import functools

import jax
import jax.numpy as jnp
from jax.experimental import pallas as pl

BS = 8
C = 3
IMG = 384
P = 16
NROWS = IMG // P
NPATCH = NROWS * NROWS
D = 384
K = 20
PP = P * P


def _fused_kernel(img_ref, w_ref, cent_ref, lab_ref, out_ref):
    img = img_ref[0]
    x = img.reshape(C * P, NROWS, P)
    patches = x.transpose(1, 0, 2).reshape(NROWS, C * P * P)
    feat = jax.lax.dot_general(
        patches, w_ref[...], (((1,), (1,)), ((), ())),
        preferred_element_type=jnp.float32)
    cent = cent_ref[...]
    c2 = jnp.sum(cent * cent, axis=1)
    dots = jax.lax.dot_general(
        feat, cent, (((1,), (1,)), ((), ())),
        preferred_element_type=jnp.float32)
    scores = c2[None, :] - 2.0 * dots
    assign = jnp.argmin(scores, axis=1)
    onehot = (assign[:, None] ==
              jax.lax.broadcasted_iota(jnp.int32, (NROWS, K), 1)
              ).astype(jnp.float32)
    preds = jax.lax.dot_general(
        onehot, lab_ref[...], (((1,), (1,)), ((), ())),
        preferred_element_type=jnp.float32)
    tiled = preds.reshape(NROWS, P, P).transpose(1, 0, 2).reshape(P, IMG)
    out_ref[0] = tiled


def kernel(image, W_patch, centroids, cluster_labels):
    grid = (BS, NROWS)
    return pl.pallas_call(
        _fused_kernel,
        grid=grid,
        in_specs=[
            pl.BlockSpec((1, C, P, IMG), lambda b, r: (b, 0, r, 0)),
            pl.BlockSpec((D, C * P * P), lambda b, r: (0, 0)),
            pl.BlockSpec((K, D), lambda b, r: (0, 0)),
            pl.BlockSpec((PP, K), lambda b, r: (0, 0)),
        ],
        out_specs=pl.BlockSpec((1, P, IMG), lambda b, r: (b, r, 0)),
        out_shape=jax.ShapeDtypeStruct((BS, IMG, IMG), jnp.float32),
    )(image, W_patch, centroids, cluster_labels)


# speedup vs baseline: 10.8888x; 2.5898x over previous
import jax
import jax.numpy as jnp
from jax.experimental import pallas as pl

BS = 8
C = 3
IMG = 384
P = 16
NROWS = IMG // P
NPATCH = NROWS * NROWS
D = 384
K = 20
PP = P * P

RB = 12


def _fused_kernel(img_ref, w_ref, cent_ref, lab_ref, out_ref):
    n = RB * NROWS
    img = img_ref[0]
    x3 = img.reshape(C * P * RB, NROWS, P)
    parts = []
    for r in range(RB):
        xr = jnp.concatenate([x3[c * RB * P + r * P:(c * RB * P + r * P) + P]
                              for c in range(C)], axis=0)
        parts.append(xr.transpose(1, 0, 2).reshape(NROWS, C * P * P))
    tokens = jnp.concatenate(parts, axis=0)
    feat = jax.lax.dot_general(
        tokens, w_ref[...], (((1,), (1,)), ((), ())),
        preferred_element_type=jnp.float32)
    cent = cent_ref[...]
    c2 = jnp.sum(cent * cent, axis=1)
    dots_t = jax.lax.dot_general(
        cent, feat, (((1,), (1,)), ((), ())),
        preferred_element_type=jnp.float32)
    scores_t = c2[:, None] - 2.0 * dots_t
    iota_t = jax.lax.broadcasted_iota(jnp.int32, (K, n), 0)
    m = jnp.min(scores_t, axis=0, keepdims=True)
    idx = jnp.min(jnp.where(scores_t == m, iota_t, K), axis=0,
                  keepdims=True)
    idx_n = jnp.transpose(idx)
    iota_n = jax.lax.broadcasted_iota(jnp.int32, (n, K), 1)
    onehot = (iota_n == idx_n).astype(jnp.bfloat16)
    lab_bf = lab_ref[...].astype(jnp.bfloat16)
    preds = jax.lax.dot_general(
        onehot, lab_bf, (((1,), (1,)), ((), ())),
        preferred_element_type=jnp.float32).astype(jnp.bfloat16)
    tiled = preds.reshape(RB, NROWS, P, P).transpose(0, 2, 1, 3)
    out_ref[0] = tiled.reshape(RB * P, IMG).astype(jnp.float32)


def kernel(image, W_patch, centroids, cluster_labels):
    steps = NROWS // RB
    return pl.pallas_call(
        _fused_kernel,
        grid=(BS, steps),
        in_specs=[
            pl.BlockSpec((1, C, RB * P, IMG), lambda b, h: (b, 0, h, 0)),
            pl.BlockSpec((D, C * P * P), lambda b, h: (0, 0)),
            pl.BlockSpec((K, D), lambda b, h: (0, 0)),
            pl.BlockSpec((PP, K), lambda b, h: (0, 0)),
        ],
        out_specs=pl.BlockSpec((1, RB * P, IMG), lambda b, h: (b, h, 0)),
        out_shape=jax.ShapeDtypeStruct((BS, IMG, IMG), jnp.float32),
    )(image, W_patch, centroids, cluster_labels)
